# Initial kernel scaffold; baseline (speedup 1.0000x reference)
import jax
import jax.numpy as jnp
from jax.experimental import pallas as pl


def kernel(x, x_neighbor, spatial, W_lr, W_hr, dec_W, dec_b, bn_gamma, bn_beta, training):
    raise NotImplementedError("write your pallas kernel here")


# speedup vs baseline: 4.0798x; 4.0798x over previous
import jax
import jax.numpy as jnp
from jax.experimental import pallas as pl
from jax.experimental.pallas import tpu as pltpu
from jax.experimental.pallas import tpu_sc as plsc

_N = 10000
_D = 64
_SCALE = 6
_K = 10
_LAT = 96
_IN_DIM = 3000
_BQ = 400
_NBLK = _N // _BQ
_NPAD = 10240
_GW = 128
_VD = 128
_NK = 10112


def _knn_body(sqq_ref, q_ref, st_ref, sqk_ref, idx_ref, d2_ref):
    qb = q_ref[...].astype(jnp.bfloat16)
    sb = st_ref[...].astype(jnp.bfloat16)
    t = jax.lax.dot_general(qb, sb, (((1,), (0,)), ((), ())),
                            preferred_element_type=jnp.float32)
    d2_ref[...] = (sqq_ref[...] + sqk_ref[...]) - 2.0 * t
    iota = jax.lax.broadcasted_iota(jnp.int32, (_BQ, _NK), 1)
    cols = []
    for _ in range(_K):
        d2 = d2_ref[...]
        m = jnp.min(d2, axis=1, keepdims=True)
        sel = jnp.where(d2 == m, iota, jnp.int32(2**30))
        j = jnp.min(sel, axis=1, keepdims=True)
        cols.append(j)
        d2_ref[...] = jnp.where(iota == j, jnp.float32(jnp.inf), d2)
    idx_ref[...] = jnp.concatenate(cols, axis=1)


def _knn(spatial):
    sq = jnp.sum(spatial * spatial, axis=1)
    st_pad = jnp.pad(spatial.T, ((0, 0), (0, _NK - _N)))
    sqk_pad = jnp.pad(sq[None, :], ((0, 0), (0, _NK - _N)),
                      constant_values=1e30)
    return pl.pallas_call(
        _knn_body,
        grid=(_NBLK,),
        in_specs=[
            pl.BlockSpec((_BQ, 1), lambda i: (i, 0)),
            pl.BlockSpec((_BQ, 2), lambda i: (i, 0)),
            pl.BlockSpec((2, _NK), lambda i: (0, 0)),
            pl.BlockSpec((1, _NK), lambda i: (0, 0)),
        ],
        out_specs=pl.BlockSpec((_BQ, _K), lambda i: (i, 0)),
        out_shape=jax.ShapeDtypeStruct((_N, _K), jnp.int32),
        scratch_shapes=[pltpu.VMEM((_BQ, _NK), jnp.float32)],
    )(sq[:, None], spatial, st_pad, sqk_pad)


def _dot(a, b):
    return jax.lax.dot_general(
        a, b, (((1,), (0,)), ((), ())),
        precision=jax.lax.Precision.HIGHEST,
        preferred_element_type=jnp.float32,
    )


def _yhr_body(xn_ref, whr_ref, out_ref):
    acc = jnp.zeros((_BQ, _LAT), jnp.float32)
    for s in range(_SCALE):
        acc = acc + _dot(xn_ref[s], whr_ref[s])
    out_ref[...] = acc


def _yhr(x_neighbor, W_hr):
    return pl.pallas_call(
        _yhr_body,
        grid=(_NBLK,),
        in_specs=[
            pl.BlockSpec((_SCALE, _BQ, _D), lambda i: (0, i, 0)),
            pl.BlockSpec((_SCALE, _D, _LAT), lambda i: (0, 0, 0)),
        ],
        out_specs=pl.BlockSpec((_BQ, _LAT), lambda i: (i, 0)),
        out_shape=jax.ShapeDtypeStruct((_N, _LAT), jnp.float32),
    )(x_neighbor.reshape(_SCALE, _N, _D), W_hr.reshape(_SCALE, _D, _LAT))


def _sc_gather(table, flat_idx):
    rows = flat_idx.shape[1]
    vd = table.shape[1]
    window = _GW
    mesh = plsc.VectorSubcoreMesh(core_axis_name="c", subcore_axis_name="s")

    @pl.kernel(out_type=jax.ShapeDtypeStruct((rows, vd), table.dtype),
               mesh=mesh)
    def gather_kernel(tab_hbm, i_hbm, o_hbm):
        def body(i_vmem, o_vmem):
            pltpu.sync_copy(tab_hbm.at[i_vmem.at[0]], o_vmem)

        pltpu.emit_pipeline(
            body,
            grid=(rows // window,),
            in_specs=[pl.BlockSpec((1, window), lambda i: (0, i))],
            out_specs=[pl.BlockSpec((window, vd), lambda i: (i, 0))],
            core_axis_name=("c", "s"),
            dimension_semantics=(pltpu.PARALLEL,),
        )(i_hbm, o_hbm)

    return gather_kernel(table, flat_idx)


def _final_body(gx_ref, gy_ref, xq_ref, wlr_ref, decw_ref, decb_ref,
                gam_ref, bet_ref, z_ref, de_ref, hr_ref):
    xq = xq_ref[...]
    fcols = []
    for k in range(_K):
        diff = gx_ref[k] - xq
        fcols.append(jnp.sum(diff * diff, axis=1, keepdims=True))
    fd2 = jnp.concatenate(fcols, axis=1)
    score = jax.nn.softmax(-fd2, axis=-1)
    agg_lr = jnp.zeros((_BQ, _VD), jnp.float32)
    agg_hr = jnp.zeros((_BQ, _VD), jnp.float32)
    for k in range(_K):
        sk = score[:, k:k + 1]
        agg_lr = agg_lr + sk * gx_ref[k]
        agg_hr = agg_hr + sk * gy_ref[k]
    z = _dot(agg_lr, wlr_ref[...])
    z_ref[...] = z
    hr_ref[...] = agg_hr[:, 0:_LAT]
    h = _dot(z, decw_ref[...]) + decb_ref[...]
    h = h / jnp.sqrt(jnp.float32(1.0 + 1e-4)) * gam_ref[...] + bet_ref[...]
    de_ref[...] = jnp.where(h > 0, h, jnp.exp(jnp.minimum(h, 0.0)) - 1.0)


def _final(gx, gy, x, W_lr, dec_W, dec_b, bn_gamma, bn_beta):
    return pl.pallas_call(
        _final_body,
        grid=(_NBLK,),
        in_specs=[
            pl.BlockSpec((_K, _BQ, _VD), lambda i: (0, i, 0)),
            pl.BlockSpec((_K, _BQ, _VD), lambda i: (0, i, 0)),
            pl.BlockSpec((_BQ, _VD), lambda i: (i, 0)),
            pl.BlockSpec((_VD, _LAT), lambda i: (0, 0)),
            pl.BlockSpec((_LAT, _IN_DIM), lambda i: (0, 0)),
            pl.BlockSpec((1, _IN_DIM), lambda i: (0, 0)),
            pl.BlockSpec((1, _IN_DIM), lambda i: (0, 0)),
            pl.BlockSpec((1, _IN_DIM), lambda i: (0, 0)),
        ],
        out_specs=[
            pl.BlockSpec((_BQ, _LAT), lambda i: (i, 0)),
            pl.BlockSpec((_BQ, _IN_DIM), lambda i: (i, 0)),
            pl.BlockSpec((_BQ, _LAT), lambda i: (i, 0)),
        ],
        out_shape=[
            jax.ShapeDtypeStruct((_N, _LAT), jnp.float32),
            jax.ShapeDtypeStruct((_N, _IN_DIM), jnp.float32),
            jax.ShapeDtypeStruct((_N, _LAT), jnp.float32),
        ],
    )(gx, gy, x, W_lr, dec_W, dec_b.reshape(1, _IN_DIM),
      bn_gamma.reshape(1, _IN_DIM), bn_beta.reshape(1, _IN_DIM))


def kernel(x, x_neighbor, spatial, W_lr, W_hr, dec_W, dec_b, bn_gamma,
           bn_beta, training):
    idx = _knn(spatial)
    yhr = _yhr(x_neighbor, W_hr)
    x_pad = jnp.pad(x, ((0, 0), (0, _VD - _D)))
    yhr_pad = jnp.pad(yhr, ((0, 0), (0, _VD - _LAT)))
    wlr_pad = jnp.pad(W_lr, ((0, _VD - _D), (0, 0)))
    idx_km = jnp.zeros((_K, _NPAD), jnp.int32).at[:, :_N].set(idx.T)
    flat = idx_km.reshape(1, _K * _NPAD)
    gx = _sc_gather(x_pad, flat).reshape(_K, _NPAD, _VD)
    gy = _sc_gather(yhr_pad, flat).reshape(_K, _NPAD, _VD)
    z, de_feat, x1_hr = _final(gx, gy, x_pad, wlr_pad, dec_W, dec_b,
                               bn_gamma, bn_beta)
    return (z, de_feat, x1_hr)


# speedup vs baseline: 4.0799x; 1.0000x over previous
import jax
import jax.numpy as jnp
from jax.experimental import pallas as pl
from jax.experimental.pallas import tpu as pltpu
from jax.experimental.pallas import tpu_sc as plsc

_N = 10000
_D = 64
_SCALE = 6
_K = 10
_LAT = 96
_IN_DIM = 3000
_BQ = 400
_NBLK = _N // _BQ
_NPAD = 10240
_GW = 128
_VD = 128
_NK = 10112


def _knn_body(sqq_ref, q_ref, st_ref, sqk_ref, idx_ref, d2_ref):
    qb = q_ref[...].astype(jnp.bfloat16)
    sb = st_ref[...].astype(jnp.bfloat16)
    t = jax.lax.dot_general(qb, sb, (((1,), (0,)), ((), ())),
                            preferred_element_type=jnp.float32)
    d2_ref[...] = (sqq_ref[...] + sqk_ref[...]) - 2.0 * t
    iota = jax.lax.broadcasted_iota(jnp.int32, (_BQ, _NK), 1)
    cols = []
    for _ in range(_K):
        d2 = d2_ref[...]
        m = jnp.min(d2, axis=1, keepdims=True)
        sel = jnp.where(d2 == m, iota, jnp.int32(2**30))
        j = jnp.min(sel, axis=1, keepdims=True)
        cols.append(j)
        d2_ref[...] = jnp.where(iota == j, jnp.float32(jnp.inf), d2)
    idx_ref[...] = jnp.concatenate(cols, axis=1)


def _knn(spatial):
    sq = jnp.sum(spatial * spatial, axis=1)
    st_pad = jnp.pad(spatial.T, ((0, 0), (0, _NK - _N)))
    sqk_pad = jnp.pad(sq[None, :], ((0, 0), (0, _NK - _N)),
                      constant_values=1e30)
    return pl.pallas_call(
        _knn_body,
        grid=(_NBLK,),
        in_specs=[
            pl.BlockSpec((_BQ, 1), lambda i: (i, 0)),
            pl.BlockSpec((_BQ, 2), lambda i: (i, 0)),
            pl.BlockSpec((2, _NK), lambda i: (0, 0)),
            pl.BlockSpec((1, _NK), lambda i: (0, 0)),
        ],
        out_specs=pl.BlockSpec((_BQ, _K), lambda i: (i, 0)),
        out_shape=jax.ShapeDtypeStruct((_N, _K), jnp.int32),
        scratch_shapes=[pltpu.VMEM((_BQ, _NK), jnp.float32)],
        compiler_params=pltpu.CompilerParams(
            dimension_semantics=("parallel",)),
    )(sq[:, None], spatial, st_pad, sqk_pad)


def _dot(a, b):
    return jax.lax.dot_general(
        a, b, (((1,), (0,)), ((), ())),
        precision=jax.lax.Precision.HIGHEST,
        preferred_element_type=jnp.float32,
    )


def _yhr_body(xn_ref, whr_ref, out_ref):
    acc = jnp.zeros((_BQ, _LAT), jnp.float32)
    for s in range(_SCALE):
        acc = acc + _dot(xn_ref[s], whr_ref[s])
    out_ref[...] = acc


def _yhr(x_neighbor, W_hr):
    return pl.pallas_call(
        _yhr_body,
        grid=(_NBLK,),
        in_specs=[
            pl.BlockSpec((_SCALE, _BQ, _D), lambda i: (0, i, 0)),
            pl.BlockSpec((_SCALE, _D, _LAT), lambda i: (0, 0, 0)),
        ],
        out_specs=pl.BlockSpec((_BQ, _LAT), lambda i: (i, 0)),
        out_shape=jax.ShapeDtypeStruct((_N, _LAT), jnp.float32),
        compiler_params=pltpu.CompilerParams(
            dimension_semantics=("parallel",)),
    )(x_neighbor.reshape(_SCALE, _N, _D), W_hr.reshape(_SCALE, _D, _LAT))


def _sc_gather(table, flat_idx):
    rows = flat_idx.shape[1]
    vd = table.shape[1]
    window = _GW
    mesh = plsc.VectorSubcoreMesh(core_axis_name="c", subcore_axis_name="s")

    @pl.kernel(out_type=jax.ShapeDtypeStruct((rows, vd), table.dtype),
               mesh=mesh)
    def gather_kernel(tab_hbm, i_hbm, o_hbm):
        def body(i_vmem, o_vmem):
            pltpu.sync_copy(tab_hbm.at[i_vmem.at[0]], o_vmem)

        pltpu.emit_pipeline(
            body,
            grid=(rows // window,),
            in_specs=[pl.BlockSpec((1, window), lambda i: (0, i))],
            out_specs=[pl.BlockSpec((window, vd), lambda i: (i, 0))],
            core_axis_name=("c", "s"),
            dimension_semantics=(pltpu.PARALLEL,),
        )(i_hbm, o_hbm)

    return gather_kernel(table, flat_idx)


def _final_body(gx_ref, gy_ref, xq_ref, wlr_ref, decw_ref, decb_ref,
                gam_ref, bet_ref, z_ref, de_ref, hr_ref):
    xq = xq_ref[...]
    fcols = []
    for k in range(_K):
        diff = gx_ref[k] - xq
        fcols.append(jnp.sum(diff * diff, axis=1, keepdims=True))
    fd2 = jnp.concatenate(fcols, axis=1)
    score = jax.nn.softmax(-fd2, axis=-1)
    agg_lr = jnp.zeros((_BQ, _VD), jnp.float32)
    agg_hr = jnp.zeros((_BQ, _VD), jnp.float32)
    for k in range(_K):
        sk = score[:, k:k + 1]
        agg_lr = agg_lr + sk * gx_ref[k]
        agg_hr = agg_hr + sk * gy_ref[k]
    z = _dot(agg_lr, wlr_ref[...])
    z_ref[...] = z
    hr_ref[...] = agg_hr[:, 0:_LAT]
    h = _dot(z, decw_ref[...]) + decb_ref[...]
    h = h / jnp.sqrt(jnp.float32(1.0 + 1e-4)) * gam_ref[...] + bet_ref[...]
    de_ref[...] = jnp.where(h > 0, h, jnp.exp(jnp.minimum(h, 0.0)) - 1.0)


def _final(gx, gy, x, W_lr, dec_W, dec_b, bn_gamma, bn_beta):
    return pl.pallas_call(
        _final_body,
        grid=(_NBLK,),
        in_specs=[
            pl.BlockSpec((_K, _BQ, _VD), lambda i: (0, i, 0)),
            pl.BlockSpec((_K, _BQ, _VD), lambda i: (0, i, 0)),
            pl.BlockSpec((_BQ, _VD), lambda i: (i, 0)),
            pl.BlockSpec((_VD, _LAT), lambda i: (0, 0)),
            pl.BlockSpec((_LAT, _IN_DIM), lambda i: (0, 0)),
            pl.BlockSpec((1, _IN_DIM), lambda i: (0, 0)),
            pl.BlockSpec((1, _IN_DIM), lambda i: (0, 0)),
            pl.BlockSpec((1, _IN_DIM), lambda i: (0, 0)),
        ],
        out_specs=[
            pl.BlockSpec((_BQ, _LAT), lambda i: (i, 0)),
            pl.BlockSpec((_BQ, _IN_DIM), lambda i: (i, 0)),
            pl.BlockSpec((_BQ, _LAT), lambda i: (i, 0)),
        ],
        out_shape=[
            jax.ShapeDtypeStruct((_N, _LAT), jnp.float32),
            jax.ShapeDtypeStruct((_N, _IN_DIM), jnp.float32),
            jax.ShapeDtypeStruct((_N, _LAT), jnp.float32),
        ],
        compiler_params=pltpu.CompilerParams(
            dimension_semantics=("parallel",)),
    )(gx, gy, x, W_lr, dec_W, dec_b.reshape(1, _IN_DIM),
      bn_gamma.reshape(1, _IN_DIM), bn_beta.reshape(1, _IN_DIM))


def kernel(x, x_neighbor, spatial, W_lr, W_hr, dec_W, dec_b, bn_gamma,
           bn_beta, training):
    idx = _knn(spatial)
    yhr = _yhr(x_neighbor, W_hr)
    x_pad = jnp.pad(x, ((0, 0), (0, _VD - _D)))
    yhr_pad = jnp.pad(yhr, ((0, 0), (0, _VD - _LAT)))
    wlr_pad = jnp.pad(W_lr, ((0, _VD - _D), (0, 0)))
    idx_km = jnp.zeros((_K, _NPAD), jnp.int32).at[:, :_N].set(idx.T)
    flat = idx_km.reshape(1, _K * _NPAD)
    gx = _sc_gather(x_pad, flat).reshape(_K, _NPAD, _VD)
    gy = _sc_gather(yhr_pad, flat).reshape(_K, _NPAD, _VD)
    z, de_feat, x1_hr = _final(gx, gy, x_pad, wlr_pad, dec_W, dec_b,
                               bn_gamma, bn_beta)
    return (z, de_feat, x1_hr)


# speedup vs baseline: 4.2678x; 1.0461x over previous
import jax
import jax.numpy as jnp
from jax.experimental import pallas as pl
from jax.experimental.pallas import tpu as pltpu
from jax.experimental.pallas import tpu_sc as plsc

_N = 10000
_D = 64
_SCALE = 6
_K = 10
_LAT = 96
_IN_DIM = 3000
_BQ = 400
_NBLK = _N // _BQ
_NPAD = 10240
_GW = 128
_VD = 128
_NK = 10112


def _knn_body(sqq_ref, q_ref, st_ref, sqk_ref, idx_ref, d2_ref):
    qb = q_ref[...].astype(jnp.bfloat16)
    sb = st_ref[...].astype(jnp.bfloat16)
    t = jax.lax.dot_general(qb, sb, (((1,), (0,)), ((), ())),
                            preferred_element_type=jnp.float32)
    d2_ref[...] = (sqq_ref[...] + sqk_ref[...]) - 2.0 * t
    iota = jax.lax.broadcasted_iota(jnp.int32, (_BQ, _NK), 1)
    cols = []
    for k in range(_K):
        d2 = d2_ref[...]
        m = jnp.min(d2, axis=1, keepdims=True)
        sel = jnp.where(d2 == m, iota, jnp.int32(2**30))
        j = jnp.min(sel, axis=1, keepdims=True)
        cols.append(j)
        if k + 1 < _K:
            d2_ref[...] = jnp.where(iota == j, jnp.float32(jnp.inf), d2)
    idx_ref[...] = jnp.concatenate(cols, axis=1)


def _knn(spatial):
    sq = jnp.sum(spatial * spatial, axis=1)
    st_pad = jnp.pad(spatial.T, ((0, 0), (0, _NK - _N)))
    sqk_pad = jnp.pad(sq[None, :], ((0, 0), (0, _NK - _N)),
                      constant_values=1e30)
    return pl.pallas_call(
        _knn_body,
        grid=(_NBLK,),
        in_specs=[
            pl.BlockSpec((_BQ, 1), lambda i: (i, 0)),
            pl.BlockSpec((_BQ, 2), lambda i: (i, 0)),
            pl.BlockSpec((2, _NK), lambda i: (0, 0)),
            pl.BlockSpec((1, _NK), lambda i: (0, 0)),
        ],
        out_specs=pl.BlockSpec((_BQ, _K), lambda i: (i, 0)),
        out_shape=jax.ShapeDtypeStruct((_N, _K), jnp.int32),
        scratch_shapes=[pltpu.VMEM((_BQ, _NK), jnp.float32)],
        compiler_params=pltpu.CompilerParams(
            dimension_semantics=("parallel",)),
    )(sq[:, None], spatial, st_pad, sqk_pad)


def _dot(a, b):
    return jax.lax.dot_general(
        a, b, (((1,), (0,)), ((), ())),
        precision=jax.lax.Precision.HIGHEST,
        preferred_element_type=jnp.float32,
    )


def _dot_bf16(a, b):
    return jax.lax.dot_general(
        a.astype(jnp.bfloat16), b.astype(jnp.bfloat16),
        (((1,), (0,)), ((), ())),
        preferred_element_type=jnp.float32,
    )


def _bf(v):
    return v.astype(jnp.bfloat16).astype(jnp.float32)


def _yhr_body(xn_ref, whr_ref, out_ref):
    acc = jnp.zeros((_BQ, _LAT), jnp.float32)
    for s in range(_SCALE):
        acc = acc + _dot(xn_ref[s], whr_ref[s])
    out_ref[...] = acc


def _yhr(x_neighbor, W_hr):
    return pl.pallas_call(
        _yhr_body,
        grid=(_NBLK,),
        in_specs=[
            pl.BlockSpec((_SCALE, _BQ, _D), lambda i: (0, i, 0)),
            pl.BlockSpec((_SCALE, _D, _LAT), lambda i: (0, 0, 0)),
        ],
        out_specs=pl.BlockSpec((_BQ, _LAT), lambda i: (i, 0)),
        out_shape=jax.ShapeDtypeStruct((_N, _LAT), jnp.float32),
        compiler_params=pltpu.CompilerParams(
            dimension_semantics=("parallel",)),
    )(x_neighbor.reshape(_SCALE, _N, _D), W_hr.reshape(_SCALE, _D, _LAT))


def _sc_gather(table, flat_idx):
    rows = flat_idx.shape[1]
    vd = table.shape[1]
    window = _GW
    mesh = plsc.VectorSubcoreMesh(core_axis_name="c", subcore_axis_name="s")

    @pl.kernel(out_type=jax.ShapeDtypeStruct((rows, vd), table.dtype),
               mesh=mesh)
    def gather_kernel(tab_hbm, i_hbm, o_hbm):
        def body(i_vmem, o_vmem):
            pltpu.sync_copy(tab_hbm.at[i_vmem.at[0]], o_vmem)

        pltpu.emit_pipeline(
            body,
            grid=(rows // window,),
            in_specs=[pl.BlockSpec((1, window), lambda i: (0, i))],
            out_specs=[pl.BlockSpec((window, vd), lambda i: (i, 0))],
            core_axis_name=("c", "s"),
            dimension_semantics=(pltpu.PARALLEL,),
        )(i_hbm, o_hbm)

    return gather_kernel(table, flat_idx)


def _final_body(gx_ref, gy_ref, xq_ref, wlr_ref, decw_ref, decb_ref,
                gam_ref, bet_ref, z_ref, de_ref, hr_ref):
    xq = xq_ref[...]
    fcols = []
    for k in range(_K):
        diff = gx_ref[k] - xq
        fcols.append(jnp.sum(diff * diff, axis=1, keepdims=True))
    fd2 = jnp.concatenate(fcols, axis=1)
    score = jax.nn.softmax(-fd2, axis=-1)
    agg_lr = jnp.zeros((_BQ, _VD), jnp.float32)
    agg_hr = jnp.zeros((_BQ, _VD), jnp.float32)
    for k in range(_K):
        sk = _bf(score[:, k:k + 1])
        agg_lr = agg_lr + sk * _bf(gx_ref[k])
        agg_hr = agg_hr + sk * _bf(gy_ref[k])
    z = _dot_bf16(agg_lr, wlr_ref[...])
    z_ref[...] = z
    hr_ref[...] = agg_hr[:, 0:_LAT]
    h = _dot_bf16(z, decw_ref[...]) + decb_ref[...]
    h = h / jnp.sqrt(jnp.float32(1.0 + 1e-4)) * gam_ref[...] + bet_ref[...]
    de_ref[...] = jnp.where(h > 0, h, jnp.exp(jnp.minimum(h, 0.0)) - 1.0)


def _final(gx, gy, x, W_lr, dec_W, dec_b, bn_gamma, bn_beta):
    return pl.pallas_call(
        _final_body,
        grid=(_NBLK,),
        in_specs=[
            pl.BlockSpec((_K, _BQ, _VD), lambda i: (0, i, 0)),
            pl.BlockSpec((_K, _BQ, _VD), lambda i: (0, i, 0)),
            pl.BlockSpec((_BQ, _VD), lambda i: (i, 0)),
            pl.BlockSpec((_VD, _LAT), lambda i: (0, 0)),
            pl.BlockSpec((_LAT, _IN_DIM), lambda i: (0, 0)),
            pl.BlockSpec((1, _IN_DIM), lambda i: (0, 0)),
            pl.BlockSpec((1, _IN_DIM), lambda i: (0, 0)),
            pl.BlockSpec((1, _IN_DIM), lambda i: (0, 0)),
        ],
        out_specs=[
            pl.BlockSpec((_BQ, _LAT), lambda i: (i, 0)),
            pl.BlockSpec((_BQ, _IN_DIM), lambda i: (i, 0)),
            pl.BlockSpec((_BQ, _LAT), lambda i: (i, 0)),
        ],
        out_shape=[
            jax.ShapeDtypeStruct((_N, _LAT), jnp.float32),
            jax.ShapeDtypeStruct((_N, _IN_DIM), jnp.float32),
            jax.ShapeDtypeStruct((_N, _LAT), jnp.float32),
        ],
        compiler_params=pltpu.CompilerParams(
            dimension_semantics=("parallel",)),
    )(gx, gy, x, W_lr, dec_W, dec_b.reshape(1, _IN_DIM),
      bn_gamma.reshape(1, _IN_DIM), bn_beta.reshape(1, _IN_DIM))


def kernel(x, x_neighbor, spatial, W_lr, W_hr, dec_W, dec_b, bn_gamma,
           bn_beta, training):
    idx = _knn(spatial)
    yhr = _yhr(x_neighbor, W_hr)
    x_pad = jnp.pad(x, ((0, 0), (0, _VD - _D)))
    yhr_pad = jnp.pad(yhr, ((0, 0), (0, _VD - _LAT)))
    wlr_pad = jnp.pad(W_lr, ((0, _VD - _D), (0, 0)))
    idx_km = jnp.zeros((_K, _NPAD), jnp.int32).at[:, :_N].set(idx.T)
    flat = idx_km.reshape(1, _K * _NPAD)
    gx = _sc_gather(x_pad, flat).reshape(_K, _NPAD, _VD)
    gy = _sc_gather(yhr_pad, flat).reshape(_K, _NPAD, _VD)
    z, de_feat, x1_hr = _final(gx, gy, x_pad, wlr_pad, dec_W, dec_b,
                               bn_gamma, bn_beta)
    return (z, de_feat, x1_hr)


# speedup vs baseline: 4.7261x; 1.1074x over previous
import jax
import jax.numpy as jnp
from jax.experimental import pallas as pl
from jax.experimental.pallas import tpu as pltpu
from jax.experimental.pallas import tpu_sc as plsc

_N = 10000
_D = 64
_SCALE = 6
_K = 10
_LAT = 96
_IN_DIM = 3000
_BQ = 200
_NBLK = _N // _BQ
_NPAD = 10240
_GW = 128
_VD = 128
_NK = 10112


def _knn_body(sqq_ref, q_ref, st_ref, sqk_ref, idx_ref, d2_ref):
    qb = q_ref[...].astype(jnp.bfloat16)
    sb = st_ref[...].astype(jnp.bfloat16)
    t = jax.lax.dot_general(qb, sb, (((1,), (0,)), ((), ())),
                            preferred_element_type=jnp.float32)
    d2_ref[...] = (sqq_ref[...] + sqk_ref[...]) - 2.0 * t
    iota = jax.lax.broadcasted_iota(jnp.int32, (_BQ, _NK), 1)
    cols = []
    for k in range(_K):
        d2 = d2_ref[...]
        m = jnp.min(d2, axis=1, keepdims=True)
        sel = jnp.where(d2 == m, iota, jnp.int32(2**30))
        j = jnp.min(sel, axis=1, keepdims=True)
        cols.append(j)
        if k + 1 < _K:
            d2_ref[...] = jnp.where(iota == j, jnp.float32(jnp.inf), d2)
    idx_ref[...] = jnp.concatenate(cols, axis=1)


def _knn(spatial):
    sq = jnp.sum(spatial * spatial, axis=1)
    st_pad = jnp.pad(spatial.T, ((0, 0), (0, _NK - _N)))
    sqk_pad = jnp.pad(sq[None, :], ((0, 0), (0, _NK - _N)),
                      constant_values=1e30)
    return pl.pallas_call(
        _knn_body,
        grid=(_NBLK,),
        in_specs=[
            pl.BlockSpec((_BQ, 1), lambda i: (i, 0)),
            pl.BlockSpec((_BQ, 2), lambda i: (i, 0)),
            pl.BlockSpec((2, _NK), lambda i: (0, 0)),
            pl.BlockSpec((1, _NK), lambda i: (0, 0)),
        ],
        out_specs=pl.BlockSpec((_BQ, _K), lambda i: (i, 0)),
        out_shape=jax.ShapeDtypeStruct((_N, _K), jnp.int32),
        scratch_shapes=[pltpu.VMEM((_BQ, _NK), jnp.float32)],
        compiler_params=pltpu.CompilerParams(
            dimension_semantics=("parallel",)),
    )(sq[:, None], spatial, st_pad, sqk_pad)


def _dot(a, b):
    return jax.lax.dot_general(
        a, b, (((1,), (0,)), ((), ())),
        precision=jax.lax.Precision.HIGHEST,
        preferred_element_type=jnp.float32,
    )


def _dot_bf16(a, b):
    return jax.lax.dot_general(
        a.astype(jnp.bfloat16), b.astype(jnp.bfloat16),
        (((1,), (0,)), ((), ())),
        preferred_element_type=jnp.float32,
    )


def _bf(v):
    return v.astype(jnp.bfloat16).astype(jnp.float32)


def _yhr_body(xn_ref, whr_ref, out_ref):
    acc = jnp.zeros((_BQ, _LAT), jnp.float32)
    for s in range(_SCALE):
        acc = acc + _dot(xn_ref[s], whr_ref[s])
    out_ref[...] = acc


def _yhr(x_neighbor, W_hr):
    return pl.pallas_call(
        _yhr_body,
        grid=(_NBLK,),
        in_specs=[
            pl.BlockSpec((_SCALE, _BQ, _D), lambda i: (0, i, 0)),
            pl.BlockSpec((_SCALE, _D, _LAT), lambda i: (0, 0, 0)),
        ],
        out_specs=pl.BlockSpec((_BQ, _LAT), lambda i: (i, 0)),
        out_shape=jax.ShapeDtypeStruct((_N, _LAT), jnp.float32),
        compiler_params=pltpu.CompilerParams(
            dimension_semantics=("parallel",)),
    )(x_neighbor.reshape(_SCALE, _N, _D), W_hr.reshape(_SCALE, _D, _LAT))


def _sc_gather(table, flat_idx):
    rows = flat_idx.shape[1]
    vd = table.shape[1]
    window = _GW
    mesh = plsc.VectorSubcoreMesh(core_axis_name="c", subcore_axis_name="s")

    @pl.kernel(out_type=jax.ShapeDtypeStruct((rows, vd), table.dtype),
               mesh=mesh)
    def gather_kernel(tab_hbm, i_hbm, o_hbm):
        def body(i_vmem, o_vmem):
            pltpu.sync_copy(tab_hbm.at[i_vmem.at[0]], o_vmem)

        pltpu.emit_pipeline(
            body,
            grid=(rows // window,),
            in_specs=[pl.BlockSpec((1, window), lambda i: (0, i))],
            out_specs=[pl.BlockSpec((window, vd), lambda i: (i, 0))],
            core_axis_name=("c", "s"),
            dimension_semantics=(pltpu.PARALLEL,),
        )(i_hbm, o_hbm)

    return gather_kernel(table, flat_idx)


def _final_body(gx_ref, gy_ref, xq_ref, wlr_ref, decw_ref, decb_ref,
                gam_ref, bet_ref, z_ref, de_ref, hr_ref):
    xq = xq_ref[...]
    fcols = []
    for k in range(_K):
        diff = gx_ref[k] - xq
        fcols.append(jnp.sum(diff * diff, axis=1, keepdims=True))
    fd2 = jnp.concatenate(fcols, axis=1)
    score = jax.nn.softmax(-fd2, axis=-1)
    agg_lr = jnp.zeros((_BQ, _VD), jnp.float32)
    agg_hr = jnp.zeros((_BQ, _VD), jnp.float32)
    for k in range(_K):
        sk = _bf(score[:, k:k + 1])
        agg_lr = agg_lr + sk * _bf(gx_ref[k])
        agg_hr = agg_hr + sk * _bf(gy_ref[k])
    z = _dot_bf16(agg_lr, wlr_ref[...])
    z_ref[...] = z
    hr_ref[...] = agg_hr[:, 0:_LAT]
    h = _dot_bf16(z, decw_ref[...]) + decb_ref[...]
    h = h / jnp.sqrt(jnp.float32(1.0 + 1e-4)) * gam_ref[...] + bet_ref[...]
    de_ref[...] = jnp.where(h > 0, h, jnp.exp(jnp.minimum(h, 0.0)) - 1.0)


def _final(gx, gy, x, W_lr, dec_W, dec_b, bn_gamma, bn_beta):
    return pl.pallas_call(
        _final_body,
        grid=(_NBLK,),
        in_specs=[
            pl.BlockSpec((_K, _BQ, _VD), lambda i: (0, i, 0)),
            pl.BlockSpec((_K, _BQ, _VD), lambda i: (0, i, 0)),
            pl.BlockSpec((_BQ, _VD), lambda i: (i, 0)),
            pl.BlockSpec((_VD, _LAT), lambda i: (0, 0)),
            pl.BlockSpec((_LAT, _IN_DIM), lambda i: (0, 0)),
            pl.BlockSpec((1, _IN_DIM), lambda i: (0, 0)),
            pl.BlockSpec((1, _IN_DIM), lambda i: (0, 0)),
            pl.BlockSpec((1, _IN_DIM), lambda i: (0, 0)),
        ],
        out_specs=[
            pl.BlockSpec((_BQ, _LAT), lambda i: (i, 0)),
            pl.BlockSpec((_BQ, _IN_DIM), lambda i: (i, 0)),
            pl.BlockSpec((_BQ, _LAT), lambda i: (i, 0)),
        ],
        out_shape=[
            jax.ShapeDtypeStruct((_N, _LAT), jnp.float32),
            jax.ShapeDtypeStruct((_N, _IN_DIM), jnp.float32),
            jax.ShapeDtypeStruct((_N, _LAT), jnp.float32),
        ],
        compiler_params=pltpu.CompilerParams(
            dimension_semantics=("parallel",)),
    )(gx, gy, x, W_lr, dec_W, dec_b.reshape(1, _IN_DIM),
      bn_gamma.reshape(1, _IN_DIM), bn_beta.reshape(1, _IN_DIM))


def kernel(x, x_neighbor, spatial, W_lr, W_hr, dec_W, dec_b, bn_gamma,
           bn_beta, training):
    idx = _knn(spatial)
    yhr = _yhr(x_neighbor, W_hr)
    x_pad = jnp.pad(x, ((0, 0), (0, _VD - _D)))
    yhr_pad = jnp.pad(yhr, ((0, 0), (0, _VD - _LAT)))
    wlr_pad = jnp.pad(W_lr, ((0, _VD - _D), (0, 0)))
    idx_km = jnp.zeros((_K, _NPAD), jnp.int32).at[:, :_N].set(idx.T)
    flat = idx_km.reshape(1, _K * _NPAD)
    gx = _sc_gather(x_pad, flat).reshape(_K, _NPAD, _VD)
    gy = _sc_gather(yhr_pad, flat).reshape(_K, _NPAD, _VD)
    z, de_feat, x1_hr = _final(gx, gy, x_pad, wlr_pad, dec_W, dec_b,
                               bn_gamma, bn_beta)
    return (z, de_feat, x1_hr)
